# baseline (speedup 1.0000x reference)
import jax
import jax.numpy as jnp
from jax import lax
from jax.experimental import pallas as pl
from jax.experimental.pallas import tpu as pltpu
from jax.experimental.pallas import tpu_sc as plsc

NC = 2
NS = 16
LANES = 16
NW = NC * NS

CHUNK = 128
NBUF = 3


def _dot16(ubuf, ibuf, row, d):
    acc = None
    for c in range(d // LANES):
        u = ubuf[row, pl.ds(c * LANES, LANES)]
        v = ibuf[row, pl.ds(c * LANES, LANES)]
        t = u * v
        acc = t if acc is None else acc + t
    return plsc.cumsum(acc)


def _cf_body(uid_hbm, iid_hbm, utab_hbm, itab_hbm, out_hbm,
             uidx_v, iidx_v, ub0, ib0, ub1, ib1, ub2, ib2, out_v,
             su0, si0, su1, si1, su2, si2):
    ubufs = [ub0, ub1, ub2]
    ibufs = [ib0, ib1, ib2]
    sems_u = [su0, su1, su2]
    sems_i = [si0, si1, si2]
    d = utab_hbm.shape[1]
    bpw = uidx_v.shape[0]
    n_chunks = bpw // CHUNK
    wid = lax.axis_index("s") * NC + lax.axis_index("c")
    base = wid * bpw

    pltpu.sync_copy(uid_hbm.at[pl.ds(base, bpw)], uidx_v)
    pltpu.sync_copy(iid_hbm.at[pl.ds(base, bpw)], iidx_v)

    lane = lax.iota(jnp.int32, LANES)
    zero16 = jnp.zeros((LANES,), jnp.int32)
    out_mask = lane == LANES - 1

    def start(j):
        s = j % NBUF
        cu = pltpu.async_copy(
            utab_hbm.at[uidx_v.at[pl.ds(j * CHUNK, CHUNK)]], ubufs[s],
            sems_u[s])
        ci = pltpu.async_copy(
            itab_hbm.at[iidx_v.at[pl.ds(j * CHUNK, CHUNK)]], ibufs[s],
            sems_i[s])
        return cu, ci

    inflight = {}
    for j in range(min(NBUF, n_chunks)):
        inflight[j] = start(j)

    for j in range(n_chunks):
        cu, ci = inflight.pop(j)
        cu.wait()
        ci.wait()
        s = j % NBUF

        @plsc.parallel_loop(0, CHUNK, unroll=2)
        def _row(r, s=s, j=j):
            scan = _dot16(ubufs[s], ibufs[s], r, d)
            idx = jnp.full((LANES,), j * CHUNK, jnp.int32) + r
            plsc.store_scatter(out_v, [idx], scan, mask=out_mask)

        if j + NBUF < n_chunks:
            inflight[j + NBUF] = start(j + NBUF)

    pltpu.sync_copy(out_v, out_hbm.at[pl.ds(base, bpw)])


def kernel(user_ids, item_ids, user_hidden_emb, item_hidden_emb):
    b = user_ids.shape[0]
    d = user_hidden_emb.shape[1]
    bpw = b // NW

    mesh = plsc.VectorSubcoreMesh(core_axis_name="c", subcore_axis_name="s")
    f = pl.kernel(
        _cf_body,
        out_type=jax.ShapeDtypeStruct((b,), jnp.float32),
        mesh=mesh,
        compiler_params=pltpu.CompilerParams(
            needs_layout_passes=False,
            skip_device_barrier=True,
            disable_bounds_checks=True,
            disable_semaphore_checks=True,
        ),
        scratch_types=[
            pltpu.VMEM((bpw,), jnp.int32),
            pltpu.VMEM((bpw,), jnp.int32),
            pltpu.VMEM((CHUNK, d), jnp.float32),
            pltpu.VMEM((CHUNK, d), jnp.float32),
            pltpu.VMEM((CHUNK, d), jnp.float32),
            pltpu.VMEM((CHUNK, d), jnp.float32),
            pltpu.VMEM((CHUNK, d), jnp.float32),
            pltpu.VMEM((CHUNK, d), jnp.float32),
            pltpu.VMEM((bpw,), jnp.float32),
            pltpu.SemaphoreType.DMA,
            pltpu.SemaphoreType.DMA,
            pltpu.SemaphoreType.DMA,
            pltpu.SemaphoreType.DMA,
            pltpu.SemaphoreType.DMA,
            pltpu.SemaphoreType.DMA,
        ],
    )
    out = f(user_ids.astype(jnp.int32), item_ids.astype(jnp.int32),
            user_hidden_emb, item_hidden_emb)
    return out.reshape(-1, 1)


# speedup vs baseline: 1.0076x; 1.0076x over previous
import jax
import jax.numpy as jnp
from jax import lax
from jax.experimental import pallas as pl
from jax.experimental.pallas import tpu as pltpu
from jax.experimental.pallas import tpu_sc as plsc

NC = 2
NS = 16
LANES = 16
NW = NC * NS

CHUNK = 128
NBUF = 3


def _dot16(ubuf, ibuf, row, d):
    acc = None
    for c in range(d // LANES):
        u = ubuf[row, pl.ds(c * LANES, LANES)]
        v = ibuf[row, pl.ds(c * LANES, LANES)]
        t = u * v
        acc = t if acc is None else acc + t
    return plsc.cumsum(acc)


def _cf_body(uid_hbm, iid_hbm, utab_hbm, itab_hbm, out_hbm,
             uidx_v, iidx_v, ub0, ib0, ub1, ib1, out_v,
             su0, si0, su1, si1):
    d = utab_hbm.shape[1]
    bpw = uidx_v.shape[0]
    n_chunks = bpw // CHUNK
    wid = lax.axis_index("s") * NC + lax.axis_index("c")
    base = wid * bpw

    pltpu.sync_copy(uid_hbm.at[pl.ds(base, bpw)], uidx_v)
    pltpu.sync_copy(iid_hbm.at[pl.ds(base, bpw)], iidx_v)

    lane = lax.iota(jnp.int32, LANES)
    out_mask = lane == LANES - 1

    def start(j, ub, ib, su, si):
        pltpu.async_copy(utab_hbm.at[uidx_v.at[pl.ds(j * CHUNK, CHUNK)]],
                         ub, su)
        pltpu.async_copy(itab_hbm.at[iidx_v.at[pl.ds(j * CHUNK, CHUNK)]],
                         ib, si)

    def phase(j, ub, ib, su, si):
        pltpu.make_async_copy(
            utab_hbm.at[uidx_v.at[pl.ds(0, CHUNK)]], ub, su).wait()
        pltpu.make_async_copy(
            itab_hbm.at[iidx_v.at[pl.ds(0, CHUNK)]], ib, si).wait()

        @plsc.parallel_loop(0, CHUNK, unroll=2)
        def _row(r):
            scan = _dot16(ub, ib, r, d)
            idx = jnp.full((LANES,), 0, jnp.int32) + (j * CHUNK + r)
            plsc.store_scatter(out_v, [idx], scan, mask=out_mask)

        @pl.when(j + 2 < n_chunks)
        def _():
            start(j + 2, ub, ib, su, si)

    start(0, ub0, ib0, su0, si0)
    start(1, ub1, ib1, su1, si1)

    def pair_body(p, carry):
        phase(2 * p, ub0, ib0, su0, si0)
        phase(2 * p + 1, ub1, ib1, su1, si1)
        return carry

    lax.fori_loop(0, n_chunks // 2, pair_body, 0)

    pltpu.sync_copy(out_v, out_hbm.at[pl.ds(base, bpw)])


def kernel(user_ids, item_ids, user_hidden_emb, item_hidden_emb):
    b = user_ids.shape[0]
    d = user_hidden_emb.shape[1]
    bpw = b // NW

    mesh = plsc.VectorSubcoreMesh(core_axis_name="c", subcore_axis_name="s")
    f = pl.kernel(
        _cf_body,
        out_type=jax.ShapeDtypeStruct((b,), jnp.float32),
        mesh=mesh,
        compiler_params=pltpu.CompilerParams(needs_layout_passes=False),
        scratch_types=[
            pltpu.VMEM((bpw,), jnp.int32),
            pltpu.VMEM((bpw,), jnp.int32),
            pltpu.VMEM((CHUNK, d), jnp.float32),
            pltpu.VMEM((CHUNK, d), jnp.float32),
            pltpu.VMEM((CHUNK, d), jnp.float32),
            pltpu.VMEM((CHUNK, d), jnp.float32),
            pltpu.VMEM((bpw,), jnp.float32),
            pltpu.SemaphoreType.DMA,
            pltpu.SemaphoreType.DMA,
            pltpu.SemaphoreType.DMA,
            pltpu.SemaphoreType.DMA,
        ],
    )
    out = f(user_ids.astype(jnp.int32), item_ids.astype(jnp.int32),
            user_hidden_emb, item_hidden_emb)
    return out.reshape(-1, 1)


# speedup vs baseline: 1.0267x; 1.0190x over previous
import jax
import jax.numpy as jnp
from jax import lax
from jax.experimental import pallas as pl
from jax.experimental.pallas import tpu as pltpu
from jax.experimental.pallas import tpu_sc as plsc

NC = 2
NS = 16
LANES = 16
NW = NC * NS

CHUNK = 128
NBUF = 3


def _dot16(ubuf, ibuf, row, d):
    acc = None
    for c in range(d // LANES):
        u = ubuf[row, pl.ds(c * LANES, LANES)]
        v = ibuf[row, pl.ds(c * LANES, LANES)]
        t = u * v
        acc = t if acc is None else acc + t
    return plsc.cumsum(acc)


def _cf_body(uid_hbm, iid_hbm, utab_hbm, itab_hbm, out_hbm,
             uidx_v, iidx_v, ub0, ib0, ub1, ib1, out_v,
             su0, si0, su1, si1, so):
    d = utab_hbm.shape[1]
    bpw = uidx_v.shape[0]
    n_chunks = bpw // CHUNK
    wid = lax.axis_index("s") * NC + lax.axis_index("c")
    base = wid * bpw

    c_u = pltpu.async_copy(uid_hbm.at[pl.ds(base, bpw)], uidx_v, su0)
    c_i = pltpu.async_copy(iid_hbm.at[pl.ds(base, bpw)], iidx_v, si0)
    c_u.wait()
    c_i.wait()

    lane = lax.iota(jnp.int32, LANES)
    out_mask = lane == LANES - 1

    def start(j, ub, ib, su, si):
        pltpu.async_copy(utab_hbm.at[uidx_v.at[pl.ds(j * CHUNK, CHUNK)]],
                         ub, su)
        pltpu.async_copy(itab_hbm.at[iidx_v.at[pl.ds(j * CHUNK, CHUNK)]],
                         ib, si)

    def phase(j, ub, ib, su, si):
        pltpu.make_async_copy(
            utab_hbm.at[uidx_v.at[pl.ds(0, CHUNK)]], ub, su).wait()
        pltpu.make_async_copy(
            itab_hbm.at[iidx_v.at[pl.ds(0, CHUNK)]], ib, si).wait()

        @plsc.parallel_loop(0, CHUNK, unroll=2)
        def _row(r):
            scan = _dot16(ub, ib, r, d)
            idx = jnp.full((LANES,), 0, jnp.int32) + (j * CHUNK + r)
            plsc.store_scatter(out_v, [idx], scan, mask=out_mask)

        @pl.when(j + 2 < n_chunks)
        def _():
            start(j + 2, ub, ib, su, si)

    start(0, ub0, ib0, su0, si0)
    start(1, ub1, ib1, su1, si1)

    def pair_body(p, carry):
        phase(2 * p, ub0, ib0, su0, si0)
        phase(2 * p + 1, ub1, ib1, su1, si1)
        pltpu.async_copy(out_v.at[pl.ds(2 * p * CHUNK, 2 * CHUNK)],
                         out_hbm.at[pl.ds(base + 2 * p * CHUNK, 2 * CHUNK)],
                         so)
        return carry

    lax.fori_loop(0, n_chunks // 2, pair_body, 0)

    for _ in range(n_chunks // 2):
        pltpu.make_async_copy(
            out_v.at[pl.ds(0, 2 * CHUNK)],
            out_hbm.at[pl.ds(base, 2 * CHUNK)], so).wait()


def kernel(user_ids, item_ids, user_hidden_emb, item_hidden_emb):
    b = user_ids.shape[0]
    d = user_hidden_emb.shape[1]
    bpw = b // NW

    mesh = plsc.VectorSubcoreMesh(core_axis_name="c", subcore_axis_name="s")
    f = pl.kernel(
        _cf_body,
        out_type=jax.ShapeDtypeStruct((b,), jnp.float32),
        mesh=mesh,
        compiler_params=pltpu.CompilerParams(needs_layout_passes=False),
        scratch_types=[
            pltpu.VMEM((bpw,), jnp.int32),
            pltpu.VMEM((bpw,), jnp.int32),
            pltpu.VMEM((CHUNK, d), jnp.float32),
            pltpu.VMEM((CHUNK, d), jnp.float32),
            pltpu.VMEM((CHUNK, d), jnp.float32),
            pltpu.VMEM((CHUNK, d), jnp.float32),
            pltpu.VMEM((bpw,), jnp.float32),
            pltpu.SemaphoreType.DMA,
            pltpu.SemaphoreType.DMA,
            pltpu.SemaphoreType.DMA,
            pltpu.SemaphoreType.DMA,
            pltpu.SemaphoreType.DMA,
        ],
    )
    out = f(user_ids.astype(jnp.int32), item_ids.astype(jnp.int32),
            user_hidden_emb, item_hidden_emb)
    return out.reshape(-1, 1)
